# baseline (speedup 1.0000x reference)
import jax
import jax.numpy as jnp
from jax import lax
from jax.experimental import pallas as pl
from jax.experimental.pallas import tpu as pltpu
from jax.experimental.pallas import tpu_sc as plsc

_INFO = plsc.get_sparse_core_info()
_NC = _INFO.num_cores
_NS = _INFO.num_subcores
_NL = _INFO.num_lanes
_NW = _NC * _NS

_B = 16384
_D = 16
_N = 1000000

_SLAB = 62464
_CHUNK = 1024
_NCHUNK = 61
_TAIL = 576

_CAP = _B
_SENT = 2**30
_HUGE = 2**30


def _scal(vec):
    return jnp.max(vec)


def _a_body(thetaT, xT, strag_all, idx_all,
            rows_out, bidx_out, cnts_out,
            idx_v, bkt_r, bkt_b, cb0, cb1, strag_v,
            tmp_r, tmp_b, mini_rows, mini_b, q_ref, nf_ref,
            sem0, sem1, semt):
    cid = lax.axis_index("c")
    sid = lax.axis_index("s")
    w = cid * _NS + sid
    iota = lax.iota(jnp.int32, _NL)

    lane_lo = sid * _SLAB
    slab_sz = jnp.where(sid == _NS - 1, _SLAB + _TAIL, _SLAB)
    lane_hi = lane_lo + slab_sz

    pltpu.sync_copy(idx_all.at[pl.ds(cid * _B, _B)], idx_v)

    hugev = jnp.full((_NL,), _HUGE, jnp.int32)
    def fill_body(j, carry):
        bkt_r[pl.ds(j * _NL, _NL)] = hugev
        return carry
    lax.fori_loop(0, (_CAP + _NL) // _NL, fill_body, 0)

    def b_body(j, cnt):
        u = idx_v[pl.ds(j * _NL, _NL)]
        m = (u >= lane_lo) & (u < lane_hi)
        plsc.store_compressed(bkt_r.at[pl.ds(cnt, _NL)], u - lane_lo, mask=m)
        plsc.store_compressed(bkt_b.at[pl.ds(cnt, _NL)], j * _NL + iota, mask=m)
        return cnt + _scal(plsc.all_reduce_population_count(m))
    cnt = lax.fori_loop(0, _B // _NL, b_body, jnp.int32(0))
    ngv = (cnt + _NL - 1) // _NL

    sentv = jnp.full((_NL,), _SENT, jnp.int32)
    def reset_mini():
        for q in range(8):
            mini_b[pl.ds(q * _NL, _NL)] = sentv
    reset_mini()
    q_ref[...] = jnp.zeros((_NL,), jnp.int32)
    nf_ref[...] = jnp.zeros((_NL,), jnp.int32)

    rows_base = w * (_CAP * _D)
    bidx_base = w * _CAP

    def extract(buf, kk, csize):
        lo_c = kk * _CHUNK
        hi_c = lo_c + csize

        def g_body(G, carry):
            rr = bkt_r[pl.ds(G * _NL, _NL)]
            m = (rr >= lo_c) & (rr < hi_c)
            nm = _scal(plsc.all_reduce_population_count(m))

            @pl.when(nm > 0)
            def _():
                bb = bkt_b[pl.ds(G * _NL, _NL)]
                plsc.store_compressed(tmp_r.at[pl.ds(0, _NL)],
                                      rr - lo_c, mask=m)
                plsc.store_compressed(tmp_b.at[pl.ds(0, _NL)], bb, mask=m)
                r_c = tmp_r[pl.ds(0, _NL)]
                b_c = tmp_b[pl.ds(0, _NL)]
                live = iota < nm
                r_c = jnp.where(live, r_c, 0)
                b_c = jnp.where(live, b_c, _SENT)

                q = _scal(q_ref[...])
                mini_b[pl.ds(q * _NL, _NL)] = b_c
                for d in range(_D):
                    dv = jnp.full((_NL,), d, jnp.int32)
                    vals = plsc.load_gather(buf, [dv, r_c])
                    mini_rows[pl.ds(q * 256 + d * _NL, _NL)] = vals

                qn = q + 1

                @pl.when(qn == 8)
                def _():
                    nf = _scal(nf_ref[...])
                    pltpu.sync_copy(
                        mini_rows,
                        rows_out.at[pl.ds(rows_base + nf * 2048, 2048)])
                    pltpu.sync_copy(
                        mini_b,
                        bidx_out.at[pl.ds(bidx_base + nf * 128, 128)])
                    reset_mini()
                    nf_ref[...] = jnp.full((_NL,), nf + 1, jnp.int32)
                    q_ref[...] = jnp.zeros((_NL,), jnp.int32)

                @pl.when(qn < 8)
                def _():
                    q_ref[...] = jnp.full((_NL,), qn, jnp.int32)

            return carry

        lax.fori_loop(0, ngv, g_body, 0)

    for c in range(_NC):
        @pl.when(cid == c)
        def _():
            src = thetaT if c == 0 else xT
            pltpu.async_copy(src.at[:, pl.ds(lane_lo, _CHUNK)], cb0, sem0)

            def k_body(k, carry):
                nxt = lane_lo + (k + 1) * _CHUNK

                @pl.when(k + 1 < _NCHUNK)
                def _():
                    @pl.when(lax.rem(k, 2) == 0)
                    def _():
                        pltpu.async_copy(
                            src.at[:, pl.ds(nxt, _CHUNK)], cb1, sem1)

                    @pl.when(lax.rem(k, 2) == 1)
                    def _():
                        pltpu.async_copy(
                            src.at[:, pl.ds(nxt, _CHUNK)], cb0, sem0)

                @pl.when(lax.rem(k, 2) == 0)
                def _():
                    pltpu.make_async_copy(
                        src.at[:, pl.ds(lane_lo, _CHUNK)], cb0, sem0).wait()
                    extract(cb0, k, _CHUNK)

                @pl.when(lax.rem(k, 2) == 1)
                def _():
                    pltpu.make_async_copy(
                        src.at[:, pl.ds(lane_lo, _CHUNK)], cb1, sem1).wait()
                    extract(cb1, k, _CHUNK)

                return carry

            lax.fori_loop(0, _NCHUNK, k_body, 0)

            @pl.when(sid == _NS - 1)
            def _():
                strag = strag_all.at[pl.ds(c * _D, _D), :]
                pltpu.async_copy(strag, strag_v, semt)
                pltpu.make_async_copy(strag, strag_v, semt).wait()
                extract(strag_v, _NCHUNK, _TAIL)

    q_fin = _scal(q_ref[...])
    nf_fin = _scal(nf_ref[...])

    @pl.when(q_fin > 0)
    def _():
        pltpu.sync_copy(
            mini_rows, rows_out.at[pl.ds(rows_base + nf_fin * 2048, 2048)])
        pltpu.sync_copy(
            mini_b, bidx_out.at[pl.ds(bidx_base + nf_fin * 128, 128)])

    ngroups = nf_fin * 8 + q_fin
    q_ref[...] = jnp.full((_NL,), ngroups, jnp.int32)
    pltpu.sync_copy(q_ref, cnts_out.at[pl.ds(w * _NL, _NL)])


def _b_body(rows_in, bidx_in, cnts_in,
            tg_out, xg_out,
            cnt_v, bloc, rloc, rowbuf, sem, sems):
    cid = lax.axis_index("c")
    sid = lax.axis_index("s")
    w = cid * _NS + sid
    iota = lax.iota(jnp.int32, _NL)

    pltpu.sync_copy(cnts_in.at[pl.ds(w * _NL, _NL)], cnt_v)
    ngroups = _scal(cnt_v[...])
    nblocks = (ngroups + 7) // 8

    rows_base = w * (_CAP * _D)
    bidx_base = w * _CAP

    for c in range(_NC):
        @pl.when(cid == c)
        def _():
            tgt = tg_out if c == 0 else xg_out

            def blk_body(blk, carry):
                pltpu.sync_copy(
                    bidx_in.at[pl.ds(bidx_base + blk * 128, 128)], bloc)
                pltpu.sync_copy(
                    rows_in.at[pl.ds(rows_base + blk * 2048, 2048)], rloc)
                for q in range(8):
                    evec = jnp.full((_NL,), q * _NL, jnp.int32) + iota
                    for d in range(_D):
                        vals = rloc[pl.ds(q * 256 + d * _NL, _NL)]
                        dv = jnp.full((_NL,), d, jnp.int32)
                        plsc.store_scatter(rowbuf, [evec, dv], vals)
                for q in range(8):
                    bq = bloc[pl.ds(q * _NL, _NL)]
                    cp = pltpu.async_copy(
                        rowbuf.at[pl.ds(q * _NL, _NL)],
                        tgt.at[plsc.Indices(bq, ignored_value=2**30)],
                        sems)
                    cp.wait()
                return carry

            lax.fori_loop(0, nblocks, blk_body, 0)


def _c_body(tg_in, xg_in, out_hbm, tg_v, xg_v, out_v, sem_t, sem_x):
    cid = lax.axis_index("c")
    sid = lax.axis_index("s")
    w = sid * _NC + cid
    base = w * (_B // _NW)
    bpw = _B // _NW
    iota = lax.iota(jnp.int32, _NL)

    ct = pltpu.async_copy(tg_in.at[pl.ds(base, bpw)], tg_v, sem_t)
    cx = pltpu.async_copy(xg_in.at[pl.ds(base, bpw)], xg_v, sem_x)
    ct.wait()
    cx.wait()

    def g_body(g, carry):
        rows = g * _NL + iota
        acc = jnp.zeros((_NL,), jnp.float32)
        for col in range(_D):
            colv = jnp.full((_NL,), col, jnp.int32)
            tv = plsc.load_gather(tg_v, [rows, colv])
            xv = plsc.load_gather(xg_v, [rows, colv])
            acc = acc + tv * xv
        out_v[pl.ds(g * _NL, _NL)] = acc
        return carry

    lax.fori_loop(0, bpw // _NL, g_body, 0)
    pltpu.sync_copy(out_v, out_hbm.at[pl.ds(base, bpw)])


@jax.jit
def _predict(theta, X, user_indices, item_indices):
    mesh = plsc.VectorSubcoreMesh(core_axis_name="c", subcore_axis_name="s")

    rows_flat, bidx_flat, cnts = pl.kernel(
        _a_body,
        out_type=(
            jax.ShapeDtypeStruct((_NW * _CAP * _D,), jnp.float32),
            jax.ShapeDtypeStruct((_NW * _CAP,), jnp.int32),
            jax.ShapeDtypeStruct((_NW * _NL,), jnp.int32),
        ),
        mesh=mesh,
        compiler_params=pltpu.CompilerParams(needs_layout_passes=False),
        scratch_types=[
            pltpu.VMEM((_B,), jnp.int32),
            pltpu.VMEM((_CAP + _NL,), jnp.int32),
            pltpu.VMEM((_CAP + _NL,), jnp.int32),
            pltpu.VMEM((_D, _CHUNK), jnp.float32),
            pltpu.VMEM((_D, _CHUNK), jnp.float32),
            pltpu.VMEM((_D, _TAIL), jnp.float32),
            pltpu.VMEM((_NL,), jnp.int32),
            pltpu.VMEM((_NL,), jnp.int32),
            pltpu.VMEM((2048,), jnp.float32),
            pltpu.VMEM((128,), jnp.int32),
            pltpu.VMEM((_NL,), jnp.int32),
            pltpu.VMEM((_NL,), jnp.int32),
            pltpu.SemaphoreType.DMA,
            pltpu.SemaphoreType.DMA,
            pltpu.SemaphoreType.DMA,
        ],
    )(theta.T, X.T,
      jnp.concatenate(
          [lax.slice(theta.T, (0, _NS * _SLAB), (_D, _N)),
           lax.slice(X.T, (0, _NS * _SLAB), (_D, _N))], axis=0),
      jnp.concatenate([user_indices, item_indices]))

    tg, xg = pl.kernel(
        _b_body,
        out_type=(
            jax.ShapeDtypeStruct((_B, _D), jnp.float32),
            jax.ShapeDtypeStruct((_B, _D), jnp.float32),
        ),
        mesh=mesh,
        compiler_params=pltpu.CompilerParams(
            needs_layout_passes=False, use_tc_tiling_on_sc=False),
        scratch_types=[
            pltpu.VMEM((_NL,), jnp.int32),
            pltpu.VMEM((128,), jnp.int32),
            pltpu.VMEM((2048,), jnp.float32),
            pltpu.VMEM((128, _D), jnp.float32),
            pltpu.SemaphoreType.DMA,
            pltpu.SemaphoreType.DMA,
        ],
    )(rows_flat, bidx_flat, cnts)

    return pl.kernel(
        _c_body,
        out_type=jax.ShapeDtypeStruct((_B,), jnp.float32),
        mesh=mesh,
        compiler_params=pltpu.CompilerParams(
            needs_layout_passes=False, use_tc_tiling_on_sc=False),
        scratch_types=[
            pltpu.VMEM((_B // _NW, _D), jnp.float32),
            pltpu.VMEM((_B // _NW, _D), jnp.float32),
            pltpu.VMEM((_B // _NW,), jnp.float32),
            pltpu.SemaphoreType.DMA,
            pltpu.SemaphoreType.DMA,
        ],
    )(tg, xg)


def kernel(theta, X, user_indices, item_indices):
    return _predict(theta, X, user_indices, item_indices)
